# Initial kernel scaffold; baseline (speedup 1.0000x reference)
import jax
import jax.numpy as jnp
from jax.experimental import pallas as pl


def kernel(X, edge_index, W0, W1):
    raise NotImplementedError("write your pallas kernel here")


# speedup vs baseline: 2.7780x; 2.7780x over previous
import functools

import jax
import jax.numpy as jnp
from jax import lax
from jax.experimental import pallas as pl
from jax.experimental.pallas import tpu as pltpu
from jax.experimental.pallas import tpu_sc as plsc

N_NODES = 10000
N_EDGES = 160000
D_IN = 256
D_H1 = 128
D_H2 = 64

_INFO = plsc.get_sparse_core_info()
NC = _INFO.num_cores
NS = _INFO.num_subcores
NW = NC * NS

N_PAD = 10240
CHUNK = 128
E_PAD = 163840


def _make_spmm(d):
    e_per_w = E_PAD // NW
    n_chunks = e_per_w // CHUNK
    rows_per_tile = N_PAD // NS
    mesh = plsc.VectorSubcoreMesh(core_axis_name="c", subcore_axis_name="s")

    @functools.partial(
        pl.kernel,
        out_type=jax.ShapeDtypeStruct((NC, N_PAD, d), jnp.float32),
        mesh=mesh,
        compiler_params=pltpu.CompilerParams(use_tc_tiling_on_sc=False),
        scratch_types=[
            pltpu.VMEM((CHUNK,), jnp.int32),
            pltpu.VMEM((CHUNK,), jnp.int32),
            pltpu.VMEM((CHUNK, d), jnp.float32),
            pltpu.VMEM_SHARED((N_PAD, d), jnp.float32),
            pltpu.SemaphoreType.DMA,
        ],
    )
    def spmm(table_hbm, src_hbm, dst_hbm, zeros_hbm, out_hbm,
             src_v, dst_v, rows_v, acc_sh, sem):
        cid = lax.axis_index("c")
        sid = lax.axis_index("s")
        wid = sid * NC + cid
        lo = sid * rows_per_tile
        pltpu.sync_copy(zeros_hbm.at[pl.ds(lo, rows_per_tile)],
                        acc_sh.at[pl.ds(lo, rows_per_tile)])
        plsc.subcore_barrier()

        base = wid * e_per_w

        def body(i, carry):
            off = base + i * CHUNK
            pltpu.sync_copy(src_hbm.at[pl.ds(off, CHUNK)], src_v)
            pltpu.sync_copy(dst_hbm.at[pl.ds(off, CHUNK)], dst_v)
            pltpu.async_copy(table_hbm.at[src_v], rows_v, sem).wait()
            pltpu.sync_copy(rows_v, acc_sh.at[dst_v], add=True)
            return carry

        lax.fori_loop(0, n_chunks, body, 0)
        plsc.subcore_barrier()
        pltpu.sync_copy(acc_sh.at[pl.ds(lo, rows_per_tile)],
                        out_hbm.at[cid, pl.ds(lo, rows_per_tile)])

    return spmm


_spmm_h1 = _make_spmm(D_H1)
_spmm_h2 = _make_spmm(D_H2)


def _mm_xw0(x, w0):
    blk = 2000

    def body(x_ref, w_ref, o_ref):
        o_ref[...] = jnp.dot(x_ref[...], w_ref[...],
                             preferred_element_type=jnp.float32)

    return pl.pallas_call(
        body,
        grid=(N_NODES // blk,),
        in_specs=[
            pl.BlockSpec((blk, D_IN), lambda i: (i, 0)),
            pl.BlockSpec((D_IN, D_H1), lambda i: (0, 0)),
        ],
        out_specs=pl.BlockSpec((blk, D_H1), lambda i: (i, 0)),
        out_shape=jax.ShapeDtypeStruct((N_NODES, D_H1), jnp.float32),
    )(x, w0)


def _relu_mm(p, w1):
    blk = 1280

    def body(p_ref, w_ref, o_ref):
        h = jnp.maximum(p_ref[0] + p_ref[1], 0.0)
        o_ref[...] = jnp.dot(h, w_ref[...], preferred_element_type=jnp.float32)

    return pl.pallas_call(
        body,
        grid=(N_PAD // blk,),
        in_specs=[
            pl.BlockSpec((NC, blk, D_H1), lambda i: (0, i, 0)),
            pl.BlockSpec((D_H1, D_H2), lambda i: (0, 0)),
        ],
        out_specs=pl.BlockSpec((blk, D_H2), lambda i: (i, 0)),
        out_shape=jax.ShapeDtypeStruct((N_PAD, D_H2), jnp.float32),
    )(p, w1)


def _decode(pz):
    blk = 400

    def body(pi_ref, pf_ref, o_ref):
        zi = pi_ref[0] + pi_ref[1]
        zf = pf_ref[0] + pf_ref[1]
        zj = zf[:N_NODES]
        logits = lax.dot_general(zi, zj, (((1,), (1,)), ((), ())),
                                 preferred_element_type=jnp.float32)
        o_ref[...] = jax.nn.sigmoid(logits)

    return pl.pallas_call(
        body,
        grid=(N_NODES // blk,),
        in_specs=[
            pl.BlockSpec((NC, blk, D_H2), lambda i: (0, i, 0)),
            pl.BlockSpec((NC, N_PAD, D_H2), lambda i: (0, 0, 0)),
        ],
        out_specs=pl.BlockSpec((blk, N_NODES), lambda i: (i, 0)),
        out_shape=jax.ShapeDtypeStruct((N_NODES, N_NODES), jnp.float32),
    )(pz, pz)


def kernel(X, edge_index, W0, W1):
    src = edge_index[0]
    dst = edge_index[1]
    pad = E_PAD - N_EDGES
    src_p = jnp.concatenate([src, jnp.zeros((pad,), jnp.int32)])
    dst_p = jnp.concatenate([dst, jnp.full((pad,), N_NODES, jnp.int32)])

    zeros1 = jnp.zeros((N_PAD, D_H1), jnp.float32)
    zeros2 = jnp.zeros((N_PAD, D_H2), jnp.float32)

    xw0 = _mm_xw0(X, W0)
    p1 = _spmm_h1(xw0, src_p, dst_p, zeros1)
    hw1 = _relu_mm(p1, W1)
    p2 = _spmm_h2(hw1, src_p, dst_p, zeros2)
    return _decode(p2)
